# baseline (speedup 1.0000x reference)
import functools

import jax
import jax.numpy as jnp
from jax import lax
from jax.experimental import pallas as pl
from jax.experimental.pallas import tpu as pltpu
from jax.experimental.pallas import tpu_sc as plsc

T = 2048
D = 1024
E = 8
K = 2
BLK = 256
NBLK = 24
NP = NBLK * BLK
NSUB = 16
CHUNK = (T * K) // NSUB


def _lane():
    return lax.iota(jnp.int32, 16)


def _lane_scalar(vec, lane, e):
    return jnp.sum(jnp.where(lane == e, vec, 0))


def _router_body(x_ref, wr_ref, logits_ref, i1_ref, i2_ref, w1_ref, w2_ref):
    x = x_ref[...]
    logits = lax.dot_general(x, wr_ref[...], (((1,), (1,)), ((), ())),
                             preferred_element_type=jnp.float32)
    logits_ref[...] = logits
    lane = lax.broadcasted_iota(jnp.int32, (T, E), 1)
    m1 = jnp.max(logits, axis=1, keepdims=True)
    i1 = jnp.min(jnp.where(logits == m1, lane, E), axis=1, keepdims=True)
    masked = jnp.where(lane == i1, -jnp.inf, logits)
    m2 = jnp.max(masked, axis=1, keepdims=True)
    i2 = jnp.min(jnp.where(masked == m2, lane, E), axis=1, keepdims=True)
    w1 = 1.0 / (1.0 + jnp.exp(m2 - m1))
    i1_ref[...] = i1
    i2_ref[...] = i2
    w1_ref[...] = w1
    w2_ref[...] = 1.0 - w1


def _router(x, w_router):
    return pl.pallas_call(
        _router_body,
        out_shape=[
            jax.ShapeDtypeStruct((T, E), jnp.float32),
            jax.ShapeDtypeStruct((T, 1), jnp.int32),
            jax.ShapeDtypeStruct((T, 1), jnp.int32),
            jax.ShapeDtypeStruct((T, 1), jnp.float32),
            jax.ShapeDtypeStruct((T, 1), jnp.float32),
        ],
    )(x, w_router)


def _dispatch_body(eid_hbm, sort_tok_hbm, inv_hbm, be_hbm,
                   eid_v, cnt_v, hist_v, inv_v, pos0_v, pos1_v,
                   tok0_v, tok1_v, zf_v, be_v, tmp16_v, hist_sh, sem):
    cid = lax.axis_index("c")
    wid = lax.axis_index("s")
    lane = _lane()
    zero16 = jnp.zeros((16,), jnp.int32)

    @pl.when(cid == 0)
    def _core0():
        pltpu.sync_copy(eid_hbm.at[pl.ds(wid * CHUNK, CHUNK)], eid_v)
        for i in range(NP // NSUB // 16):
            zf_v[pl.ds(i * 16, 16)] = zero16
        pltpu.sync_copy(zf_v, sort_tok_hbm.at[pl.ds(wid * (NP // NSUB),
                                                    NP // NSUB)])

        cnt = zero16
        for j in range(CHUNK // 16):
            v = eid_v[pl.ds(j * 16, 16)]
            for e in range(E):
                c = jnp.sum(jnp.where(v == e, 1, 0))
                cnt = cnt + jnp.where(lane == e, c, 0)
        cnt_v[...] = cnt
        pltpu.sync_copy(cnt_v, hist_sh.at[wid])
        plsc.subcore_barrier()
        pltpu.sync_copy(hist_sh, hist_v)

        tot = zero16
        pre = zero16
        for w in range(NSUB):
            hv = hist_v[w, :]
            tot = tot + hv
            pre = pre + jnp.where(jnp.full((16,), w, jnp.int32) < wid, hv, 0)
        padded = ((tot + (BLK - 1)) >> 8) << 8
        csum = plsc.cumsum(padded)
        base = csum - padded
        off = base + pre
        total_padded = jnp.sum(padded)

        b_lo = lane * BLK
        b_hi = (lane + 16) * BLK
        acc_lo = jnp.full((16,), -1, jnp.int32)
        acc_hi = jnp.full((16,), -1, jnp.int32)
        for e in range(E):
            be_s = _lane_scalar(base, lane, e)
            acc_lo = acc_lo + jnp.where(b_lo >= be_s, 1, 0)
            acc_hi = acc_hi + jnp.where(b_hi >= be_s, 1, 0)
        be_v[pl.ds(0, 16)] = jnp.where(b_lo < total_padded, acc_lo, -1)
        be_v[pl.ds(16, 16)] = jnp.where(b_hi < total_padded, acc_hi, -1)

        @pl.when(wid == 0)
        def _write_be():
            pltpu.sync_copy(be_v, be_hbm)

        kk = jnp.where(wid >= (NSUB // 2), 1, 0)
        tbase = wid * CHUNK - kk * T
        running = off
        for j in range(CHUNK // 16):
            v = eid_v[pl.ds(j * 16, 16)]
            tmp16_v[...] = running
            base_l = plsc.load_gather(tmp16_v, [v])
            r = zero16
            for e in range(E):
                m = v == e
                mi = jnp.where(m, 1, 0)
                cs = plsc.cumsum(mi)
                r = r + jnp.where(m, cs - 1, 0)
                running = running + jnp.where(lane == e, jnp.sum(mi), 0)
            pos = base_l + r
            tok = tbase + j * 16 + lane
            inv_v[pl.ds(j * 16, 16)] = pos
            if j < (CHUNK // 32):
                pos0_v[pl.ds(j * 16, 16)] = pos
                tok0_v[pl.ds(j * 16, 16)] = tok
            else:
                jj = j - CHUNK // 32
                pos1_v[pl.ds(jj * 16, 16)] = pos
                tok1_v[pl.ds(jj * 16, 16)] = tok

        pltpu.sync_copy(inv_v, inv_hbm.at[pl.ds(wid * CHUNK, CHUNK)])
        pltpu.async_copy(tok0_v, sort_tok_hbm.at[pos0_v], sem).wait()
        pltpu.async_copy(tok1_v, sort_tok_hbm.at[pos1_v], sem).wait()


def _dispatch(eid_flat):
    mesh = plsc.VectorSubcoreMesh(core_axis_name="c", subcore_axis_name="s")
    f = pl.kernel(
        _dispatch_body,
        out_type=[
            jax.ShapeDtypeStruct((NP,), jnp.int32),
            jax.ShapeDtypeStruct((T * K,), jnp.int32),
            jax.ShapeDtypeStruct((32,), jnp.int32),
        ],
        mesh=mesh,
        compiler_params=pltpu.CompilerParams(needs_layout_passes=False),
        scratch_types=[
            pltpu.VMEM((CHUNK,), jnp.int32),
            pltpu.VMEM((16,), jnp.int32),
            pltpu.VMEM((NSUB, 16), jnp.int32),
            pltpu.VMEM((CHUNK,), jnp.int32),
            pltpu.VMEM((CHUNK // 2,), jnp.int32),
            pltpu.VMEM((CHUNK // 2,), jnp.int32),
            pltpu.VMEM((CHUNK // 2,), jnp.int32),
            pltpu.VMEM((CHUNK // 2,), jnp.int32),
            pltpu.VMEM((NP // NSUB,), jnp.int32),
            pltpu.VMEM((32,), jnp.int32),
            pltpu.VMEM((16,), jnp.int32),
            pltpu.HBM((NSUB, 16), jnp.int32),
            pltpu.SemaphoreType.DMA,
        ],
    )
    return f(eid_flat)


def _gather_body(x_hbm, st_hbm, xs_hbm, idx_v, rows_v, sem):
    wid = lax.axis_index("s") * 2 + lax.axis_index("c")
    rows_per = NP // 32
    half = rows_per // 2
    for j in range(2):
        row0 = wid * rows_per + j * half
        pltpu.sync_copy(st_hbm.at[pl.ds(row0, half)], idx_v)
        for i in range(half // 16):
            u = idx_v[pl.ds(i * 16, 16)]
            idx_v[pl.ds(i * 16, 16)] = jnp.clip(u, 0, T - 1)
        pltpu.async_copy(x_hbm.at[idx_v], rows_v, sem).wait()
        pltpu.sync_copy(rows_v, xs_hbm.at[pl.ds(row0, half)])


def _gather(x, sort_tok):
    mesh = plsc.VectorSubcoreMesh(core_axis_name="c", subcore_axis_name="s")
    f = pl.kernel(
        _gather_body,
        out_type=jax.ShapeDtypeStruct((NP, D), jnp.float32),
        mesh=mesh,
        compiler_params=pltpu.CompilerParams(needs_layout_passes=False),
        scratch_types=[
            pltpu.VMEM((NP // 64,), jnp.int32),
            pltpu.VMEM((NP // 64, D), jnp.float32),
            pltpu.SemaphoreType.DMA,
        ],
    )
    return f(x, sort_tok)


def _gemm_body(be_ref, xs_ref, w13_ref, w2_ref, y_ref):
    b = pl.program_id(0)
    e = be_ref[b]

    @pl.when(e >= 0)
    def _compute():
        x = xs_ref[...]
        h = lax.dot_general(x, w13_ref[0], (((1,), (1,)), ((), ())),
                            preferred_element_type=jnp.float32)
        gate = h[:, :D]
        up = h[:, D:]
        act = gate * lax.logistic(gate) * up
        y_ref[...] = lax.dot_general(act, w2_ref[0], (((1,), (1,)), ((), ())),
                                     preferred_element_type=jnp.float32)


def _gemm(block_expert, x_s, w13, w2):
    grid_spec = pltpu.PrefetchScalarGridSpec(
        num_scalar_prefetch=1,
        grid=(NBLK,),
        in_specs=[
            pl.BlockSpec((BLK, D), lambda b, be: (b, 0)),
            pl.BlockSpec((1, 2 * D, D),
                         lambda b, be: (jnp.maximum(be[b], 0), 0, 0)),
            pl.BlockSpec((1, D, D),
                         lambda b, be: (jnp.maximum(be[b], 0), 0, 0)),
        ],
        out_specs=pl.BlockSpec((BLK, D), lambda b, be: (b, 0)),
    )
    return pl.pallas_call(
        _gemm_body,
        grid_spec=grid_spec,
        out_shape=jax.ShapeDtypeStruct((NP, D), jnp.float32),
    )(block_expert, x_s, w13, w2)


def _combine_body(inv_hbm, w0_hbm, w1_hbm, y_hbm, out_hbm,
                  i00_v, i01_v, i10_v, i11_v, w0_v, w1_v, a_v, b_v, sem):
    wid = lax.axis_index("s") * 2 + lax.axis_index("c")
    t0 = wid * (T // 32)
    pltpu.sync_copy(inv_hbm.at[pl.ds(t0, 32)], i00_v)
    pltpu.sync_copy(inv_hbm.at[pl.ds(t0 + 32, 32)], i01_v)
    pltpu.sync_copy(inv_hbm.at[pl.ds(T + t0, 32)], i10_v)
    pltpu.sync_copy(inv_hbm.at[pl.ds(T + t0 + 32, 32)], i11_v)
    pltpu.sync_copy(w0_hbm.at[pl.ds(t0, 64)], w0_v)
    pltpu.sync_copy(w1_hbm.at[pl.ds(t0, 64)], w1_v)
    for c in range(2):
        ia = i00_v if c == 0 else i01_v
        ib = i10_v if c == 0 else i11_v
        pltpu.async_copy(y_hbm.at[ia], a_v, sem).wait()
        pltpu.async_copy(y_hbm.at[ib], b_v, sem).wait()

        def _token(i, _):
            ti = c * 32 + i
            li = jnp.full((16,), ti, jnp.int32)
            s0 = plsc.load_gather(w0_v, [li])
            s1 = plsc.load_gather(w1_v, [li])

            def _col(j, _):
                cj = pl.multiple_of(j * 16, 16)
                av = a_v[i, pl.ds(cj, 16)]
                bv = b_v[i, pl.ds(cj, 16)]
                a_v[i, pl.ds(cj, 16)] = s0 * av + s1 * bv
                return 0

            lax.fori_loop(0, D // 16, _col, 0)
            return 0

        lax.fori_loop(0, 32, _token, 0)
        pltpu.sync_copy(a_v, out_hbm.at[pl.ds(t0 + c * 32, 32)])


def _combine(inv, w0, w1, y):
    mesh = plsc.VectorSubcoreMesh(core_axis_name="c", subcore_axis_name="s")
    f = pl.kernel(
        _combine_body,
        out_type=jax.ShapeDtypeStruct((T, D), jnp.float32),
        mesh=mesh,
        compiler_params=pltpu.CompilerParams(needs_layout_passes=False),
        scratch_types=[
            pltpu.VMEM((32,), jnp.int32),
            pltpu.VMEM((32,), jnp.int32),
            pltpu.VMEM((32,), jnp.int32),
            pltpu.VMEM((32,), jnp.int32),
            pltpu.VMEM((64,), jnp.float32),
            pltpu.VMEM((64,), jnp.float32),
            pltpu.VMEM((32, D), jnp.float32),
            pltpu.VMEM((32, D), jnp.float32),
            pltpu.SemaphoreType.DMA,
        ],
    )
    return f(inv, w0, w1, y)


def kernel(hidden_states, w_router, w13, w2):
    logits, i1, i2, wt1, wt2 = _router(hidden_states, w_router)
    eid_flat = jnp.concatenate([i1, i2], axis=0).reshape(T * K)
    sort_tok, inv, block_expert = _dispatch(eid_flat)
    x_s = _gather(hidden_states, sort_tok)
    y = _gemm(block_expert[:NBLK], x_s, w13, w2)
    out = _combine(inv, wt1.reshape(T), wt2.reshape(T), y)
    return out, logits


# speedup vs baseline: 1.0045x; 1.0045x over previous
import functools

import jax
import jax.numpy as jnp
from jax import lax
from jax.experimental import pallas as pl
from jax.experimental.pallas import tpu as pltpu
from jax.experimental.pallas import tpu_sc as plsc

T = 2048
D = 1024
E = 8
K = 2
BLK = 256
NBLK = 24
NP = NBLK * BLK
NSUB = 16
CHUNK = (T * K) // NSUB


def _lane():
    return lax.iota(jnp.int32, 16)


def _lane_scalar(vec, lane, e):
    return jnp.sum(jnp.where(lane == e, vec, 0))


def _router_body(x_ref, wr_ref, logits_ref, i1_ref, i2_ref, w1_ref, w2_ref):
    x = x_ref[...]
    logits = lax.dot_general(x, wr_ref[...], (((1,), (1,)), ((), ())),
                             preferred_element_type=jnp.float32)
    logits_ref[...] = logits
    lane = lax.broadcasted_iota(jnp.int32, (T, E), 1)
    m1 = jnp.max(logits, axis=1, keepdims=True)
    i1 = jnp.min(jnp.where(logits == m1, lane, E), axis=1, keepdims=True)
    masked = jnp.where(lane == i1, -jnp.inf, logits)
    m2 = jnp.max(masked, axis=1, keepdims=True)
    i2 = jnp.min(jnp.where(masked == m2, lane, E), axis=1, keepdims=True)
    w1 = 1.0 / (1.0 + jnp.exp(m2 - m1))
    i1_ref[...] = i1
    i2_ref[...] = i2
    w1_ref[...] = w1
    w2_ref[...] = 1.0 - w1


def _router(x, w_router):
    return pl.pallas_call(
        _router_body,
        out_shape=[
            jax.ShapeDtypeStruct((T, E), jnp.float32),
            jax.ShapeDtypeStruct((T, 1), jnp.int32),
            jax.ShapeDtypeStruct((T, 1), jnp.int32),
            jax.ShapeDtypeStruct((T, 1), jnp.float32),
            jax.ShapeDtypeStruct((T, 1), jnp.float32),
        ],
    )(x, w_router)


def _dispatch_body(eid_hbm, sort_tok_hbm, inv_hbm, be_hbm,
                   eid_v, cnt_v, hist_v, inv_v, pos0_v, pos1_v,
                   tok0_v, tok1_v, zf_v, be_v, tmp16_v, hist_sh, sem):
    cid = lax.axis_index("c")
    wid = lax.axis_index("s")
    lane = _lane()
    zero16 = jnp.zeros((16,), jnp.int32)

    @pl.when(cid == 0)
    def _core0():
        pltpu.sync_copy(eid_hbm.at[pl.ds(wid * CHUNK, CHUNK)], eid_v)
        for i in range(NP // NSUB // 16):
            zf_v[pl.ds(i * 16, 16)] = zero16
        pltpu.sync_copy(zf_v, sort_tok_hbm.at[pl.ds(wid * (NP // NSUB),
                                                    NP // NSUB)])

        cnt = zero16
        for j in range(CHUNK // 16):
            v = eid_v[pl.ds(j * 16, 16)]
            for e in range(E):
                c = jnp.sum(jnp.where(v == e, 1, 0))
                cnt = cnt + jnp.where(lane == e, c, 0)
        cnt_v[...] = cnt
        pltpu.sync_copy(cnt_v, hist_sh.at[wid])
        plsc.subcore_barrier()
        pltpu.sync_copy(hist_sh, hist_v)

        tot = zero16
        pre = zero16
        for w in range(NSUB):
            hv = hist_v[w, :]
            tot = tot + hv
            pre = pre + jnp.where(jnp.full((16,), w, jnp.int32) < wid, hv, 0)
        padded = ((tot + (BLK - 1)) >> 8) << 8
        csum = plsc.cumsum(padded)
        base = csum - padded
        off = base + pre
        total_padded = jnp.sum(padded)

        b_lo = lane * BLK
        b_hi = (lane + 16) * BLK
        acc_lo = jnp.full((16,), -1, jnp.int32)
        acc_hi = jnp.full((16,), -1, jnp.int32)
        for e in range(E):
            be_s = _lane_scalar(base, lane, e)
            acc_lo = acc_lo + jnp.where(b_lo >= be_s, 1, 0)
            acc_hi = acc_hi + jnp.where(b_hi >= be_s, 1, 0)
        be_v[pl.ds(0, 16)] = jnp.where(b_lo < total_padded, acc_lo, -1)
        be_v[pl.ds(16, 16)] = jnp.where(b_hi < total_padded, acc_hi, -1)

        @pl.when(wid == 0)
        def _write_be():
            pltpu.sync_copy(be_v, be_hbm)

        kk = jnp.where(wid >= (NSUB // 2), 1, 0)
        tbase = wid * CHUNK - kk * T
        running = off
        for j in range(CHUNK // 16):
            v = eid_v[pl.ds(j * 16, 16)]
            tmp16_v[...] = running
            base_l = plsc.load_gather(tmp16_v, [v])
            r = zero16
            for e in range(E):
                m = v == e
                mi = jnp.where(m, 1, 0)
                cs = plsc.cumsum(mi)
                r = r + jnp.where(m, cs - 1, 0)
                running = running + jnp.where(lane == e, jnp.sum(mi), 0)
            pos = base_l + r
            tok = tbase + j * 16 + lane
            inv_v[pl.ds(j * 16, 16)] = pos
            if j < (CHUNK // 32):
                pos0_v[pl.ds(j * 16, 16)] = pos
                tok0_v[pl.ds(j * 16, 16)] = tok
            else:
                jj = j - CHUNK // 32
                pos1_v[pl.ds(jj * 16, 16)] = pos
                tok1_v[pl.ds(jj * 16, 16)] = tok

        pltpu.sync_copy(inv_v, inv_hbm.at[pl.ds(wid * CHUNK, CHUNK)])
        pltpu.async_copy(tok0_v, sort_tok_hbm.at[pos0_v], sem).wait()
        pltpu.async_copy(tok1_v, sort_tok_hbm.at[pos1_v], sem).wait()


def _dispatch(eid_flat):
    mesh = plsc.VectorSubcoreMesh(core_axis_name="c", subcore_axis_name="s")
    f = pl.kernel(
        _dispatch_body,
        out_type=[
            jax.ShapeDtypeStruct((NP,), jnp.int32),
            jax.ShapeDtypeStruct((T * K,), jnp.int32),
            jax.ShapeDtypeStruct((32,), jnp.int32),
        ],
        mesh=mesh,
        compiler_params=pltpu.CompilerParams(needs_layout_passes=False),
        scratch_types=[
            pltpu.VMEM((CHUNK,), jnp.int32),
            pltpu.VMEM((16,), jnp.int32),
            pltpu.VMEM((NSUB, 16), jnp.int32),
            pltpu.VMEM((CHUNK,), jnp.int32),
            pltpu.VMEM((CHUNK // 2,), jnp.int32),
            pltpu.VMEM((CHUNK // 2,), jnp.int32),
            pltpu.VMEM((CHUNK // 2,), jnp.int32),
            pltpu.VMEM((CHUNK // 2,), jnp.int32),
            pltpu.VMEM((NP // NSUB,), jnp.int32),
            pltpu.VMEM((32,), jnp.int32),
            pltpu.VMEM((16,), jnp.int32),
            pltpu.HBM((NSUB, 16), jnp.int32),
            pltpu.SemaphoreType.DMA,
        ],
    )
    return f(eid_flat)


_GCH = 48
_GN = (NP // 32) // _GCH


def _gather_body(x_hbm, st_hbm, xs_hbm, idx_v, a_v, b_v, gsem_a, gsem_b,
                 wsem_a, wsem_b):
    wid = lax.axis_index("s") * 2 + lax.axis_index("c")
    rows_per = NP // 32
    base = wid * rows_per
    pltpu.sync_copy(st_hbm.at[pl.ds(base, rows_per)], idx_v)
    for i in range(rows_per // 16):
        u = idx_v[pl.ds(i * 16, 16)]
        idx_v[pl.ds(i * 16, 16)] = jnp.clip(u, 0, T - 1)

    bufs = (a_v, b_v)
    gsems = (gsem_a, gsem_b)
    wsems = (wsem_a, wsem_b)

    def gfire(c):
        return pltpu.async_copy(
            x_hbm.at[idx_v.at[pl.ds(c * _GCH, _GCH)]], bufs[c % 2],
            gsems[c % 2])

    g = [None] * _GN
    w = [None] * _GN
    g[0] = gfire(0)
    g[1] = gfire(1)
    for c in range(_GN):
        g[c].wait()
        w[c] = pltpu.async_copy(
            bufs[c % 2], xs_hbm.at[pl.ds(base + c * _GCH, _GCH)],
            wsems[c % 2])
        if c + 2 < _GN:
            w[c].wait()
            g[c + 2] = gfire(c + 2)
    w[_GN - 2].wait()
    w[_GN - 1].wait()


def _gather(x, sort_tok):
    mesh = plsc.VectorSubcoreMesh(core_axis_name="c", subcore_axis_name="s")
    f = pl.kernel(
        _gather_body,
        out_type=jax.ShapeDtypeStruct((NP, D), jnp.float32),
        mesh=mesh,
        compiler_params=pltpu.CompilerParams(needs_layout_passes=False),
        scratch_types=[
            pltpu.VMEM((NP // 32,), jnp.int32),
            pltpu.VMEM((_GCH, D), jnp.float32),
            pltpu.VMEM((_GCH, D), jnp.float32),
            pltpu.SemaphoreType.DMA,
            pltpu.SemaphoreType.DMA,
            pltpu.SemaphoreType.DMA,
            pltpu.SemaphoreType.DMA,
        ],
    )
    return f(x, sort_tok)


def _gemm_body(be_ref, xs_ref, w13_ref, w2_ref, y_ref):
    b = pl.program_id(0)
    e = be_ref[b]

    @pl.when(e >= 0)
    def _compute():
        x = xs_ref[...]
        h = lax.dot_general(x, w13_ref[0], (((1,), (1,)), ((), ())),
                            preferred_element_type=jnp.float32)
        gate = h[:, :D]
        up = h[:, D:]
        act = gate * lax.logistic(gate) * up
        y_ref[...] = lax.dot_general(act, w2_ref[0], (((1,), (1,)), ((), ())),
                                     preferred_element_type=jnp.float32)


def _gemm(block_expert, x_s, w13, w2):
    grid_spec = pltpu.PrefetchScalarGridSpec(
        num_scalar_prefetch=1,
        grid=(NBLK,),
        in_specs=[
            pl.BlockSpec((BLK, D), lambda b, be: (b, 0)),
            pl.BlockSpec((1, 2 * D, D),
                         lambda b, be: (jnp.maximum(be[b], 0), 0, 0)),
            pl.BlockSpec((1, D, D),
                         lambda b, be: (jnp.maximum(be[b], 0), 0, 0)),
        ],
        out_specs=pl.BlockSpec((BLK, D), lambda b, be: (b, 0)),
    )
    return pl.pallas_call(
        _gemm_body,
        grid_spec=grid_spec,
        out_shape=jax.ShapeDtypeStruct((NP, D), jnp.float32),
    )(block_expert, x_s, w13, w2)


def _combine_body(inv_hbm, w0_hbm, w1_hbm, y_hbm, out_hbm,
                  i00_v, i01_v, i10_v, i11_v, w0_v, w1_v, a_v, b_v, sem):
    wid = lax.axis_index("s") * 2 + lax.axis_index("c")
    t0 = wid * (T // 32)
    pltpu.sync_copy(inv_hbm.at[pl.ds(t0, 32)], i00_v)
    pltpu.sync_copy(inv_hbm.at[pl.ds(t0 + 32, 32)], i01_v)
    pltpu.sync_copy(inv_hbm.at[pl.ds(T + t0, 32)], i10_v)
    pltpu.sync_copy(inv_hbm.at[pl.ds(T + t0 + 32, 32)], i11_v)
    pltpu.sync_copy(w0_hbm.at[pl.ds(t0, 64)], w0_v)
    pltpu.sync_copy(w1_hbm.at[pl.ds(t0, 64)], w1_v)
    for c in range(2):
        ia = i00_v if c == 0 else i01_v
        ib = i10_v if c == 0 else i11_v
        pltpu.async_copy(y_hbm.at[ia], a_v, sem).wait()
        pltpu.async_copy(y_hbm.at[ib], b_v, sem).wait()

        def _token(i, _):
            ti = c * 32 + i
            li = jnp.full((16,), ti, jnp.int32)
            s0 = plsc.load_gather(w0_v, [li])
            s1 = plsc.load_gather(w1_v, [li])

            def _col(j, _):
                cj = pl.multiple_of(j * 16, 16)
                av = a_v[i, pl.ds(cj, 16)]
                bv = b_v[i, pl.ds(cj, 16)]
                a_v[i, pl.ds(cj, 16)] = s0 * av + s1 * bv
                return 0

            lax.fori_loop(0, D // 16, _col, 0)
            return 0

        lax.fori_loop(0, 32, _token, 0)
        pltpu.sync_copy(a_v, out_hbm.at[pl.ds(t0 + c * 32, 32)])


def _combine(inv, w0, w1, y):
    mesh = plsc.VectorSubcoreMesh(core_axis_name="c", subcore_axis_name="s")
    f = pl.kernel(
        _combine_body,
        out_type=jax.ShapeDtypeStruct((T, D), jnp.float32),
        mesh=mesh,
        compiler_params=pltpu.CompilerParams(needs_layout_passes=False),
        scratch_types=[
            pltpu.VMEM((32,), jnp.int32),
            pltpu.VMEM((32,), jnp.int32),
            pltpu.VMEM((32,), jnp.int32),
            pltpu.VMEM((32,), jnp.int32),
            pltpu.VMEM((64,), jnp.float32),
            pltpu.VMEM((64,), jnp.float32),
            pltpu.VMEM((32, D), jnp.float32),
            pltpu.VMEM((32, D), jnp.float32),
            pltpu.SemaphoreType.DMA,
        ],
    )
    return f(inv, w0, w1, y)


def kernel(hidden_states, w_router, w13, w2):
    logits, i1, i2, wt1, wt2 = _router(hidden_states, w_router)
    eid_flat = jnp.concatenate([i1, i2], axis=0).reshape(T * K)
    sort_tok, inv, block_expert = _dispatch(eid_flat)
    x_s = _gather(hidden_states, sort_tok)
    y = _gemm(block_expert[:NBLK], x_s, w13, w2)
    out = _combine(inv, wt1.reshape(T), wt2.reshape(T), y)
    return out, logits


# speedup vs baseline: 1.2240x; 1.2185x over previous
import functools

import jax
import jax.numpy as jnp
from jax import lax
from jax.experimental import pallas as pl
from jax.experimental.pallas import tpu as pltpu
from jax.experimental.pallas import tpu_sc as plsc

T = 2048
D = 1024
E = 8
K = 2
BLK = 256
NBLK = 24
NP = NBLK * BLK
NSUB = 16
CHUNK = (T * K) // NSUB


def _lane():
    return lax.iota(jnp.int32, 16)


def _lane_scalar(vec, lane, e):
    return jnp.sum(jnp.where(lane == e, vec, 0))


def _router_body(x_ref, wr_ref, logits_ref, i1_ref, i2_ref, w1_ref, w2_ref):
    x = x_ref[...]
    logits = lax.dot_general(x, wr_ref[...], (((1,), (1,)), ((), ())),
                             preferred_element_type=jnp.float32)
    logits_ref[...] = logits
    lane = lax.broadcasted_iota(jnp.int32, (T, E), 1)
    m1 = jnp.max(logits, axis=1, keepdims=True)
    i1 = jnp.min(jnp.where(logits == m1, lane, E), axis=1, keepdims=True)
    masked = jnp.where(lane == i1, -jnp.inf, logits)
    m2 = jnp.max(masked, axis=1, keepdims=True)
    i2 = jnp.min(jnp.where(masked == m2, lane, E), axis=1, keepdims=True)
    w1 = 1.0 / (1.0 + jnp.exp(m2 - m1))
    i1_ref[...] = i1
    i2_ref[...] = i2
    w1_ref[...] = w1
    w2_ref[...] = 1.0 - w1


def _router(x, w_router):
    return pl.pallas_call(
        _router_body,
        out_shape=[
            jax.ShapeDtypeStruct((T, E), jnp.float32),
            jax.ShapeDtypeStruct((T, 1), jnp.int32),
            jax.ShapeDtypeStruct((T, 1), jnp.int32),
            jax.ShapeDtypeStruct((T, 1), jnp.float32),
            jax.ShapeDtypeStruct((T, 1), jnp.float32),
        ],
    )(x, w_router)


def _dispatch_body(eid_hbm, sort_tok_hbm, inv_hbm, be_hbm,
                   eid_v, cnt_v, hist_v, inv_v, pos0_v, pos1_v,
                   tok0_v, tok1_v, zf_v, be_v, tmp16_v, hist_sh, sem):
    cid = lax.axis_index("c")
    wid = lax.axis_index("s")
    lane = _lane()
    zero16 = jnp.zeros((16,), jnp.int32)

    @pl.when(cid == 0)
    def _core0():
        pltpu.sync_copy(eid_hbm.at[pl.ds(wid * CHUNK, CHUNK)], eid_v)
        for i in range(NP // NSUB // 16):
            zf_v[pl.ds(i * 16, 16)] = zero16
        pltpu.sync_copy(zf_v, sort_tok_hbm.at[pl.ds(wid * (NP // NSUB),
                                                    NP // NSUB)])

        cnt = zero16
        for j in range(CHUNK // 16):
            v = eid_v[pl.ds(j * 16, 16)]
            for e in range(E):
                c = jnp.sum(jnp.where(v == e, 1, 0))
                cnt = cnt + jnp.where(lane == e, c, 0)
        cnt_v[...] = cnt
        pltpu.sync_copy(cnt_v, hist_sh.at[wid])
        plsc.subcore_barrier()
        pltpu.sync_copy(hist_sh, hist_v)

        tot = zero16
        pre = zero16
        for w in range(NSUB):
            hv = hist_v[w, :]
            tot = tot + hv
            pre = pre + jnp.where(jnp.full((16,), w, jnp.int32) < wid, hv, 0)
        padded = ((tot + (BLK - 1)) >> 8) << 8
        csum = plsc.cumsum(padded)
        base = csum - padded
        off = base + pre
        total_padded = jnp.sum(padded)

        b_lo = lane * BLK
        b_hi = (lane + 16) * BLK
        acc_lo = jnp.full((16,), -1, jnp.int32)
        acc_hi = jnp.full((16,), -1, jnp.int32)
        for e in range(E):
            be_s = _lane_scalar(base, lane, e)
            acc_lo = acc_lo + jnp.where(b_lo >= be_s, 1, 0)
            acc_hi = acc_hi + jnp.where(b_hi >= be_s, 1, 0)
        be_v[pl.ds(0, 16)] = jnp.where(b_lo < total_padded, acc_lo, -1)
        be_v[pl.ds(16, 16)] = jnp.where(b_hi < total_padded, acc_hi, -1)

        @pl.when(wid == 0)
        def _write_be():
            pltpu.sync_copy(be_v, be_hbm)

        kk = jnp.where(wid >= (NSUB // 2), 1, 0)
        tbase = wid * CHUNK - kk * T
        running = off
        for j in range(CHUNK // 16):
            v = eid_v[pl.ds(j * 16, 16)]
            tmp16_v[...] = running
            base_l = plsc.load_gather(tmp16_v, [v])
            r = zero16
            for e in range(E):
                m = v == e
                mi = jnp.where(m, 1, 0)
                cs = plsc.cumsum(mi)
                r = r + jnp.where(m, cs - 1, 0)
                running = running + jnp.where(lane == e, jnp.sum(mi), 0)
            pos = base_l + r
            tok = tbase + j * 16 + lane
            inv_v[pl.ds(j * 16, 16)] = pos
            if j < (CHUNK // 32):
                pos0_v[pl.ds(j * 16, 16)] = pos
                tok0_v[pl.ds(j * 16, 16)] = tok
            else:
                jj = j - CHUNK // 32
                pos1_v[pl.ds(jj * 16, 16)] = pos
                tok1_v[pl.ds(jj * 16, 16)] = tok

        pltpu.sync_copy(inv_v, inv_hbm.at[pl.ds(wid * CHUNK, CHUNK)])
        pltpu.async_copy(tok0_v, sort_tok_hbm.at[pos0_v], sem).wait()
        pltpu.async_copy(tok1_v, sort_tok_hbm.at[pos1_v], sem).wait()


def _dispatch(eid_flat):
    mesh = plsc.VectorSubcoreMesh(core_axis_name="c", subcore_axis_name="s")
    f = pl.kernel(
        _dispatch_body,
        out_type=[
            jax.ShapeDtypeStruct((NP,), jnp.int32),
            jax.ShapeDtypeStruct((T * K,), jnp.int32),
            jax.ShapeDtypeStruct((32,), jnp.int32),
        ],
        mesh=mesh,
        compiler_params=pltpu.CompilerParams(needs_layout_passes=False),
        scratch_types=[
            pltpu.VMEM((CHUNK,), jnp.int32),
            pltpu.VMEM((16,), jnp.int32),
            pltpu.VMEM((NSUB, 16), jnp.int32),
            pltpu.VMEM((CHUNK,), jnp.int32),
            pltpu.VMEM((CHUNK // 2,), jnp.int32),
            pltpu.VMEM((CHUNK // 2,), jnp.int32),
            pltpu.VMEM((CHUNK // 2,), jnp.int32),
            pltpu.VMEM((CHUNK // 2,), jnp.int32),
            pltpu.VMEM((NP // NSUB,), jnp.int32),
            pltpu.VMEM((32,), jnp.int32),
            pltpu.VMEM((16,), jnp.int32),
            pltpu.HBM((NSUB, 16), jnp.int32),
            pltpu.SemaphoreType.DMA,
        ],
    )
    return f(eid_flat)


_GCH = 48
_GN = (NP // 32) // _GCH


def _gather_body(x_hbm, st_hbm, be_hbm, xs_hbm, idx_v, be_v, a_v, b_v,
                 gsem_a, gsem_b, wsem_a, wsem_b):
    wid = lax.axis_index("s") * 2 + lax.axis_index("c")
    rows_per = NP // 32
    base = wid * rows_per
    pltpu.sync_copy(be_hbm, be_v)
    nact = (jnp.sum(jnp.where(be_v[pl.ds(0, 16)] >= 0, 1, 0))
            + jnp.sum(jnp.where(be_v[pl.ds(16, 16)] >= 0, 1, 0)))
    total = nact * BLK
    pltpu.sync_copy(st_hbm.at[pl.ds(base, rows_per)], idx_v)
    for i in range(rows_per // 16):
        u = idx_v[pl.ds(i * 16, 16)]
        idx_v[pl.ds(i * 16, 16)] = jnp.clip(u, 0, T - 1)

    bufs = (a_v, b_v)
    gsems = (gsem_a, gsem_b)
    wsems = (wsem_a, wsem_b)

    def act(c):
        return base + c * _GCH < total

    def gfire(c):
        @pl.when(act(c))
        def _():
            pltpu.async_copy(
                x_hbm.at[idx_v.at[pl.ds(c * _GCH, _GCH)]], bufs[c % 2],
                gsems[c % 2])

    gfire(0)
    gfire(1)
    for c in range(_GN):
        @pl.when(act(c))
        def _step(c=c):
            pltpu.make_async_copy(
                x_hbm.at[idx_v.at[pl.ds(c * _GCH, _GCH)]], bufs[c % 2],
                gsems[c % 2]).wait()
            pltpu.async_copy(
                bufs[c % 2], xs_hbm.at[pl.ds(base + c * _GCH, _GCH)],
                wsems[c % 2]).wait()
        if c + 2 < _GN:
            gfire(c + 2)


def _gather(x, sort_tok, block_expert):
    mesh = plsc.VectorSubcoreMesh(core_axis_name="c", subcore_axis_name="s")
    f = pl.kernel(
        _gather_body,
        out_type=jax.ShapeDtypeStruct((NP, D), jnp.float32),
        mesh=mesh,
        compiler_params=pltpu.CompilerParams(needs_layout_passes=False),
        scratch_types=[
            pltpu.VMEM((NP // 32,), jnp.int32),
            pltpu.VMEM((32,), jnp.int32),
            pltpu.VMEM((_GCH, D), jnp.float32),
            pltpu.VMEM((_GCH, D), jnp.float32),
            pltpu.SemaphoreType.DMA,
            pltpu.SemaphoreType.DMA,
            pltpu.SemaphoreType.DMA,
            pltpu.SemaphoreType.DMA,
        ],
    )
    return f(x, sort_tok, block_expert)


def _gemm_body(be_ref, xs_ref, w13_ref, w2_ref, y_ref):
    b = pl.program_id(0)
    e = be_ref[b]

    @pl.when(e >= 0)
    def _compute():
        x = xs_ref[...].astype(jnp.bfloat16)
        h = lax.dot_general(x, w13_ref[0].astype(jnp.bfloat16),
                            (((1,), (1,)), ((), ())),
                            preferred_element_type=jnp.float32)
        gate = h[:, :D]
        up = h[:, D:]
        act = (gate * lax.logistic(gate) * up).astype(jnp.bfloat16)
        y_ref[...] = lax.dot_general(act, w2_ref[0].astype(jnp.bfloat16),
                                     (((1,), (1,)), ((), ())),
                                     preferred_element_type=jnp.float32)


def _gemm(block_expert, x_s, w13, w2):
    grid_spec = pltpu.PrefetchScalarGridSpec(
        num_scalar_prefetch=1,
        grid=(NBLK,),
        in_specs=[
            pl.BlockSpec((BLK, D), lambda b, be: (b, 0)),
            pl.BlockSpec((1, 2 * D, D),
                         lambda b, be: (jnp.maximum(be[b], 0), 0, 0)),
            pl.BlockSpec((1, D, D),
                         lambda b, be: (jnp.maximum(be[b], 0), 0, 0)),
        ],
        out_specs=pl.BlockSpec((BLK, D), lambda b, be: (b, 0)),
    )
    return pl.pallas_call(
        _gemm_body,
        grid_spec=grid_spec,
        out_shape=jax.ShapeDtypeStruct((NP, D), jnp.float32),
    )(block_expert, x_s, w13, w2)


def _combine_body(inv_hbm, w0_hbm, w1_hbm, y_hbm, out_hbm,
                  i00_v, i01_v, i10_v, i11_v, w0_v, w1_v, a_v, b_v, sem):
    wid = lax.axis_index("s") * 2 + lax.axis_index("c")
    t0 = wid * (T // 32)
    pltpu.sync_copy(inv_hbm.at[pl.ds(t0, 32)], i00_v)
    pltpu.sync_copy(inv_hbm.at[pl.ds(t0 + 32, 32)], i01_v)
    pltpu.sync_copy(inv_hbm.at[pl.ds(T + t0, 32)], i10_v)
    pltpu.sync_copy(inv_hbm.at[pl.ds(T + t0 + 32, 32)], i11_v)
    pltpu.sync_copy(w0_hbm.at[pl.ds(t0, 64)], w0_v)
    pltpu.sync_copy(w1_hbm.at[pl.ds(t0, 64)], w1_v)
    for c in range(2):
        ia = i00_v if c == 0 else i01_v
        ib = i10_v if c == 0 else i11_v
        pltpu.async_copy(y_hbm.at[ia], a_v, sem).wait()
        pltpu.async_copy(y_hbm.at[ib], b_v, sem).wait()

        def _token(i, _):
            ti = c * 32 + i
            li = jnp.full((16,), ti, jnp.int32)
            s0 = plsc.load_gather(w0_v, [li])
            s1 = plsc.load_gather(w1_v, [li])

            def _col(j, _):
                cj = pl.multiple_of(j * 16, 16)
                av = a_v[i, pl.ds(cj, 16)]
                bv = b_v[i, pl.ds(cj, 16)]
                a_v[i, pl.ds(cj, 16)] = s0 * av + s1 * bv
                return 0

            lax.fori_loop(0, D // 16, _col, 0)
            return 0

        lax.fori_loop(0, 32, _token, 0)
        pltpu.sync_copy(a_v, out_hbm.at[pl.ds(t0 + c * 32, 32)])


def _combine(inv, w0, w1, y):
    mesh = plsc.VectorSubcoreMesh(core_axis_name="c", subcore_axis_name="s")
    f = pl.kernel(
        _combine_body,
        out_type=jax.ShapeDtypeStruct((T, D), jnp.float32),
        mesh=mesh,
        compiler_params=pltpu.CompilerParams(needs_layout_passes=False),
        scratch_types=[
            pltpu.VMEM((32,), jnp.int32),
            pltpu.VMEM((32,), jnp.int32),
            pltpu.VMEM((32,), jnp.int32),
            pltpu.VMEM((32,), jnp.int32),
            pltpu.VMEM((64,), jnp.float32),
            pltpu.VMEM((64,), jnp.float32),
            pltpu.VMEM((32, D), jnp.float32),
            pltpu.VMEM((32, D), jnp.float32),
            pltpu.SemaphoreType.DMA,
        ],
    )
    return f(inv, w0, w1, y)


def kernel(hidden_states, w_router, w13, w2):
    logits, i1, i2, wt1, wt2 = _router(hidden_states, w_router)
    eid_flat = jnp.concatenate([i1, i2], axis=0).reshape(T * K)
    sort_tok, inv, block_expert = _dispatch(eid_flat)
    x_s = _gather(hidden_states, sort_tok, block_expert)
    y = _gemm(block_expert[:NBLK], x_s, w13, w2)
    out = _combine(inv, wt1.reshape(T), wt2.reshape(T), y)
    return out, logits


# speedup vs baseline: 1.4072x; 1.1496x over previous
import functools

import jax
import jax.numpy as jnp
from jax import lax
from jax.experimental import pallas as pl
from jax.experimental.pallas import tpu as pltpu
from jax.experimental.pallas import tpu_sc as plsc

T = 2048
D = 1024
E = 8
K = 2
BLK = 256
NBLK = 24
NP = NBLK * BLK
NSUB = 16
CHUNK = (T * K) // NSUB


def _lane():
    return lax.iota(jnp.int32, 16)


def _lane_scalar(vec, lane, e):
    return jnp.sum(jnp.where(lane == e, vec, 0))


def _router_body(x_ref, wr_ref, logits_ref, i1_ref, i2_ref, w1_ref, w2_ref):
    x = x_ref[...]
    logits = lax.dot_general(x, wr_ref[...], (((1,), (1,)), ((), ())),
                             preferred_element_type=jnp.float32)
    logits_ref[...] = logits
    lane = lax.broadcasted_iota(jnp.int32, (T, E), 1)
    m1 = jnp.max(logits, axis=1, keepdims=True)
    i1 = jnp.min(jnp.where(logits == m1, lane, E), axis=1, keepdims=True)
    masked = jnp.where(lane == i1, -jnp.inf, logits)
    m2 = jnp.max(masked, axis=1, keepdims=True)
    i2 = jnp.min(jnp.where(masked == m2, lane, E), axis=1, keepdims=True)
    w1 = 1.0 / (1.0 + jnp.exp(m2 - m1))
    i1_ref[...] = i1
    i2_ref[...] = i2
    w1_ref[...] = w1
    w2_ref[...] = 1.0 - w1


def _router(x, w_router):
    return pl.pallas_call(
        _router_body,
        out_shape=[
            jax.ShapeDtypeStruct((T, E), jnp.float32),
            jax.ShapeDtypeStruct((T, 1), jnp.int32),
            jax.ShapeDtypeStruct((T, 1), jnp.int32),
            jax.ShapeDtypeStruct((T, 1), jnp.float32),
            jax.ShapeDtypeStruct((T, 1), jnp.float32),
        ],
    )(x, w_router)


_GCH = 32
_GN = (NP // 32) // _GCH


def _dg_body(eid_hbm, x_hbm, xs_hbm, inv_hbm, be_hbm,
             eid_v, cnt_v, hist_v, inv_v, pos0_v, pos1_v, tok0_v, tok1_v,
             be_v, tmp16_v, idx_v, a_v, b_v, c_v, hist_hbm, st_hbm,
             ssem, gsa, gsb, gsc, wsa, wsb, wsc):
    cid = lax.axis_index("c")
    sid = lax.axis_index("s")
    lane = _lane()
    zero16 = jnp.zeros((16,), jnp.int32)

    pltpu.sync_copy(eid_hbm.at[pl.ds(sid * CHUNK, CHUNK)], eid_v)

    cnt = zero16
    for j in range(CHUNK // 16):
        v = eid_v[pl.ds(j * 16, 16)]
        for e in range(E):
            c = jnp.sum(jnp.where(v == e, 1, 0))
            cnt = cnt + jnp.where(lane == e, c, 0)
    cnt_v[...] = cnt
    pltpu.sync_copy(cnt_v, hist_hbm.at[sid])
    plsc.subcore_barrier()
    pltpu.sync_copy(hist_hbm, hist_v)

    tot = zero16
    pre = zero16
    for w in range(NSUB):
        hv = hist_v[w, :]
        tot = tot + hv
        pre = pre + jnp.where(jnp.full((16,), w, jnp.int32) < sid, hv, 0)
    padded = ((tot + (BLK - 1)) >> 8) << 8
    csum = plsc.cumsum(padded)
    base = csum - padded
    off = base + pre
    total_padded = jnp.sum(padded)

    b_lo = lane * BLK
    b_hi = (lane + 16) * BLK
    acc_lo = jnp.full((16,), -1, jnp.int32)
    acc_hi = jnp.full((16,), -1, jnp.int32)
    for e in range(E):
        be_s = _lane_scalar(base, lane, e)
        acc_lo = acc_lo + jnp.where(b_lo >= be_s, 1, 0)
        acc_hi = acc_hi + jnp.where(b_hi >= be_s, 1, 0)
    be_v[pl.ds(0, 16)] = jnp.where(b_lo < total_padded, acc_lo, -1)
    be_v[pl.ds(16, 16)] = jnp.where(b_hi < total_padded, acc_hi, -1)

    @pl.when(jnp.logical_and(sid == 0, cid == 0))
    def _write_be():
        pltpu.sync_copy(be_v, be_hbm)

    kk = jnp.where(sid >= (NSUB // 2), 1, 0)
    tbase = sid * CHUNK - kk * T
    running = off
    for j in range(CHUNK // 16):
        v = eid_v[pl.ds(j * 16, 16)]
        tmp16_v[...] = running
        base_l = plsc.load_gather(tmp16_v, [v])
        r = zero16
        for e in range(E):
            m = v == e
            mi = jnp.where(m, 1, 0)
            cs = plsc.cumsum(mi)
            r = r + jnp.where(m, cs - 1, 0)
            running = running + jnp.where(lane == e, jnp.sum(mi), 0)
        pos = base_l + r
        tok = tbase + j * 16 + lane
        inv_v[pl.ds(j * 16, 16)] = pos
        if j < (CHUNK // 32):
            pos0_v[pl.ds(j * 16, 16)] = pos
            tok0_v[pl.ds(j * 16, 16)] = tok
        else:
            jj = j - CHUNK // 32
            pos1_v[pl.ds(jj * 16, 16)] = pos
            tok1_v[pl.ds(jj * 16, 16)] = tok

    @pl.when(cid == 0)
    def _write_inv():
        pltpu.sync_copy(inv_v, inv_hbm.at[pl.ds(sid * CHUNK, CHUNK)])

    pltpu.async_copy(tok0_v, st_hbm.at[pos0_v], ssem).wait()
    pltpu.async_copy(tok1_v, st_hbm.at[pos1_v], ssem).wait()
    plsc.subcore_barrier()

    wid = sid * 2 + cid
    rows_per = NP // 32
    gbase = wid * rows_per
    pltpu.sync_copy(st_hbm.at[pl.ds(gbase, rows_per)], idx_v)
    for i in range(rows_per // 16):
        u = idx_v[pl.ds(i * 16, 16)]
        idx_v[pl.ds(i * 16, 16)] = jnp.clip(u, 0, T - 1)

    bufs = (a_v, b_v, c_v)
    gsems = (gsa, gsb, gsc)
    wsems = (wsa, wsb, wsc)

    def gact(c):
        return gbase + c * _GCH < total_padded

    def gfire(c):
        @pl.when(gact(c))
        def _():
            pltpu.async_copy(
                x_hbm.at[idx_v.at[pl.ds(c * _GCH, _GCH)]], bufs[c % 3],
                gsems[c % 3])

    gfire(0)
    gfire(1)
    gfire(2)
    for c in range(_GN):
        @pl.when(gact(c))
        def _step(c=c):
            pltpu.make_async_copy(
                x_hbm.at[idx_v.at[pl.ds(c * _GCH, _GCH)]], bufs[c % 3],
                gsems[c % 3]).wait()
            pltpu.async_copy(
                bufs[c % 3], xs_hbm.at[pl.ds(gbase + c * _GCH, _GCH)],
                wsems[c % 3]).wait()
        if c + 3 < _GN:
            gfire(c + 3)


def _dispatch_gather(eid_flat, x):
    mesh = plsc.VectorSubcoreMesh(core_axis_name="c", subcore_axis_name="s")
    f = pl.kernel(
        _dg_body,
        out_type=[
            jax.ShapeDtypeStruct((NP, D), jnp.float32),
            jax.ShapeDtypeStruct((T * K,), jnp.int32),
            jax.ShapeDtypeStruct((32,), jnp.int32),
        ],
        mesh=mesh,
        compiler_params=pltpu.CompilerParams(needs_layout_passes=False),
        scratch_types=[
            pltpu.VMEM((CHUNK,), jnp.int32),
            pltpu.VMEM((16,), jnp.int32),
            pltpu.VMEM((NSUB, 16), jnp.int32),
            pltpu.VMEM((CHUNK,), jnp.int32),
            pltpu.VMEM((CHUNK // 2,), jnp.int32),
            pltpu.VMEM((CHUNK // 2,), jnp.int32),
            pltpu.VMEM((CHUNK // 2,), jnp.int32),
            pltpu.VMEM((CHUNK // 2,), jnp.int32),
            pltpu.VMEM((32,), jnp.int32),
            pltpu.VMEM((16,), jnp.int32),
            pltpu.VMEM((NP // 32,), jnp.int32),
            pltpu.VMEM((_GCH, D), jnp.float32),
            pltpu.VMEM((_GCH, D), jnp.float32),
            pltpu.VMEM((_GCH, D), jnp.float32),
            pltpu.HBM((NSUB, 16), jnp.int32),
            pltpu.HBM((NP,), jnp.int32),
            pltpu.SemaphoreType.DMA,
            pltpu.SemaphoreType.DMA,
            pltpu.SemaphoreType.DMA,
            pltpu.SemaphoreType.DMA,
            pltpu.SemaphoreType.DMA,
            pltpu.SemaphoreType.DMA,
            pltpu.SemaphoreType.DMA,
        ],
    )
    return f(eid_flat, x)


def _gemm_body(be_ref, xs_ref, w13_ref, w2_ref, y_ref):
    b = pl.program_id(0)
    e = be_ref[b]

    @pl.when(e >= 0)
    def _compute():
        x = xs_ref[...].astype(jnp.bfloat16)
        h = lax.dot_general(x, w13_ref[0].astype(jnp.bfloat16),
                            (((1,), (1,)), ((), ())),
                            preferred_element_type=jnp.float32)
        gate = h[:, :D]
        up = h[:, D:]
        act = (gate * lax.logistic(gate) * up).astype(jnp.bfloat16)
        y_ref[...] = lax.dot_general(act, w2_ref[0].astype(jnp.bfloat16),
                                     (((1,), (1,)), ((), ())),
                                     preferred_element_type=jnp.float32)


def _gemm(block_expert, x_s, w13, w2):
    grid_spec = pltpu.PrefetchScalarGridSpec(
        num_scalar_prefetch=1,
        grid=(NBLK,),
        in_specs=[
            pl.BlockSpec((BLK, D), lambda b, be: (b, 0)),
            pl.BlockSpec((1, 2 * D, D),
                         lambda b, be: (jnp.maximum(be[b], 0), 0, 0)),
            pl.BlockSpec((1, D, D),
                         lambda b, be: (jnp.maximum(be[b], 0), 0, 0)),
        ],
        out_specs=pl.BlockSpec((BLK, D), lambda b, be: (b, 0)),
    )
    return pl.pallas_call(
        _gemm_body,
        grid_spec=grid_spec,
        out_shape=jax.ShapeDtypeStruct((NP, D), jnp.float32),
    )(block_expert, x_s, w13, w2)


def _combine_body(inv_hbm, w0_hbm, w1_hbm, y_hbm, out_hbm,
                  i00_v, i01_v, i10_v, i11_v, w0_v, w1_v, a_v, b_v, sem):
    wid = lax.axis_index("s") * 2 + lax.axis_index("c")
    t0 = wid * (T // 32)
    pltpu.sync_copy(inv_hbm.at[pl.ds(t0, 32)], i00_v)
    pltpu.sync_copy(inv_hbm.at[pl.ds(t0 + 32, 32)], i01_v)
    pltpu.sync_copy(inv_hbm.at[pl.ds(T + t0, 32)], i10_v)
    pltpu.sync_copy(inv_hbm.at[pl.ds(T + t0 + 32, 32)], i11_v)
    pltpu.sync_copy(w0_hbm.at[pl.ds(t0, 64)], w0_v)
    pltpu.sync_copy(w1_hbm.at[pl.ds(t0, 64)], w1_v)
    for c in range(2):
        ia = i00_v if c == 0 else i01_v
        ib = i10_v if c == 0 else i11_v
        pltpu.async_copy(y_hbm.at[ia], a_v, sem).wait()
        pltpu.async_copy(y_hbm.at[ib], b_v, sem).wait()

        def _token(i, _):
            ti = c * 32 + i
            li = jnp.full((16,), ti, jnp.int32)
            s0 = plsc.load_gather(w0_v, [li])
            s1 = plsc.load_gather(w1_v, [li])

            def _col(j, _):
                cj = pl.multiple_of(j * 16, 16)
                av = a_v[i, pl.ds(cj, 16)]
                bv = b_v[i, pl.ds(cj, 16)]
                a_v[i, pl.ds(cj, 16)] = s0 * av + s1 * bv
                return 0

            lax.fori_loop(0, D // 16, _col, 0)
            return 0

        lax.fori_loop(0, 32, _token, 0)
        pltpu.sync_copy(a_v, out_hbm.at[pl.ds(t0 + c * 32, 32)])


def _combine(inv, w0, w1, y):
    mesh = plsc.VectorSubcoreMesh(core_axis_name="c", subcore_axis_name="s")
    f = pl.kernel(
        _combine_body,
        out_type=jax.ShapeDtypeStruct((T, D), jnp.float32),
        mesh=mesh,
        compiler_params=pltpu.CompilerParams(needs_layout_passes=False),
        scratch_types=[
            pltpu.VMEM((32,), jnp.int32),
            pltpu.VMEM((32,), jnp.int32),
            pltpu.VMEM((32,), jnp.int32),
            pltpu.VMEM((32,), jnp.int32),
            pltpu.VMEM((64,), jnp.float32),
            pltpu.VMEM((64,), jnp.float32),
            pltpu.VMEM((32, D), jnp.float32),
            pltpu.VMEM((32, D), jnp.float32),
            pltpu.SemaphoreType.DMA,
        ],
    )
    return f(inv, w0, w1, y)


def kernel(hidden_states, w_router, w13, w2):
    logits, i1, i2, wt1, wt2 = _router(hidden_states, w_router)
    eid_flat = jnp.concatenate([i1, i2], axis=0).reshape(T * K)
    x_s, inv, block_expert = _dispatch_gather(eid_flat, hidden_states)
    y = _gemm(block_expert[:NBLK], x_s, w13, w2)
    out = _combine(inv, wt1.reshape(T), wt2.reshape(T), y)
    return out, logits


# speedup vs baseline: 1.4477x; 1.0288x over previous
import functools

import jax
import jax.numpy as jnp
from jax import lax
from jax.experimental import pallas as pl
from jax.experimental.pallas import tpu as pltpu
from jax.experimental.pallas import tpu_sc as plsc

T = 2048
D = 1024
E = 8
K = 2
BLK = 256
NBLK = 24
NP = NBLK * BLK
NSUB = 16
CHUNK = (T * K) // NSUB


def _lane():
    return lax.iota(jnp.int32, 16)


def _lane_scalar(vec, lane, e):
    return jnp.sum(jnp.where(lane == e, vec, 0))


def _router_body(x_ref, wr_ref, logits_ref, i1_ref, i2_ref, w1_ref, w2_ref):
    x = x_ref[...]
    logits = lax.dot_general(x, wr_ref[...], (((1,), (1,)), ((), ())),
                             preferred_element_type=jnp.float32)
    logits_ref[...] = logits
    lane = lax.broadcasted_iota(jnp.int32, (T, E), 1)
    m1 = jnp.max(logits, axis=1, keepdims=True)
    i1 = jnp.min(jnp.where(logits == m1, lane, E), axis=1, keepdims=True)
    masked = jnp.where(lane == i1, -jnp.inf, logits)
    m2 = jnp.max(masked, axis=1, keepdims=True)
    i2 = jnp.min(jnp.where(masked == m2, lane, E), axis=1, keepdims=True)
    w1 = 1.0 / (1.0 + jnp.exp(m2 - m1))
    i1_ref[...] = i1
    i2_ref[...] = i2
    w1_ref[...] = w1
    w2_ref[...] = 1.0 - w1


def _router(x, w_router):
    return pl.pallas_call(
        _router_body,
        out_shape=[
            jax.ShapeDtypeStruct((T, E), jnp.float32),
            jax.ShapeDtypeStruct((T, 1), jnp.int32),
            jax.ShapeDtypeStruct((T, 1), jnp.int32),
            jax.ShapeDtypeStruct((T, 1), jnp.float32),
            jax.ShapeDtypeStruct((T, 1), jnp.float32),
        ],
    )(x, w_router)


_GCH = 32
_GN = (NP // 32) // _GCH


def _dg_body(eid_hbm, x_hbm, xs_hbm, inv_hbm, be_hbm,
             eid_v, cnt_v, hist_v, inv_v, pos0_v, pos1_v, tok0_v, tok1_v,
             be_v, tmp16_v, idx_v, a_v, b_v, c_v, hist_hbm, st_hbm,
             ssem, gsa, gsb, gsc, wsa, wsb, wsc):
    cid = lax.axis_index("c")
    sid = lax.axis_index("s")
    lane = _lane()
    zero16 = jnp.zeros((16,), jnp.int32)

    pltpu.sync_copy(eid_hbm.at[pl.ds(sid * CHUNK, CHUNK)], eid_v)

    cnt = zero16
    for j in range(CHUNK // 16):
        v = eid_v[pl.ds(j * 16, 16)]
        for e in range(E):
            c = jnp.sum(jnp.where(v == e, 1, 0))
            cnt = cnt + jnp.where(lane == e, c, 0)
    cnt_v[...] = cnt
    pltpu.sync_copy(cnt_v, hist_hbm.at[sid])
    plsc.subcore_barrier()
    pltpu.sync_copy(hist_hbm, hist_v)

    tot = zero16
    pre = zero16
    for w in range(NSUB):
        hv = hist_v[w, :]
        tot = tot + hv
        pre = pre + jnp.where(jnp.full((16,), w, jnp.int32) < sid, hv, 0)
    padded = ((tot + (BLK - 1)) >> 8) << 8
    csum = plsc.cumsum(padded)
    base = csum - padded
    off = base + pre
    total_padded = jnp.sum(padded)

    b_lo = lane * BLK
    b_hi = (lane + 16) * BLK
    acc_lo = jnp.full((16,), -1, jnp.int32)
    acc_hi = jnp.full((16,), -1, jnp.int32)
    for e in range(E):
        be_s = _lane_scalar(base, lane, e)
        acc_lo = acc_lo + jnp.where(b_lo >= be_s, 1, 0)
        acc_hi = acc_hi + jnp.where(b_hi >= be_s, 1, 0)
    be_v[pl.ds(0, 16)] = jnp.where(b_lo < total_padded, acc_lo, -1)
    be_v[pl.ds(16, 16)] = jnp.where(b_hi < total_padded, acc_hi, -1)

    @pl.when(jnp.logical_and(sid == 0, cid == 0))
    def _write_be():
        pltpu.sync_copy(be_v, be_hbm)

    kk = jnp.where(sid >= (NSUB // 2), 1, 0)
    tbase = sid * CHUNK - kk * T
    running = off
    for j in range(CHUNK // 16):
        v = eid_v[pl.ds(j * 16, 16)]
        tmp16_v[...] = running
        base_l = plsc.load_gather(tmp16_v, [v])
        r = zero16
        for e in range(E):
            m = v == e
            mi = jnp.where(m, 1, 0)
            cs = plsc.cumsum(mi)
            r = r + jnp.where(m, cs - 1, 0)
            running = running + jnp.where(lane == e, jnp.sum(mi), 0)
        pos = base_l + r
        tok = tbase + j * 16 + lane
        inv_v[pl.ds(j * 16, 16)] = pos
        posc = pos + cid * NP
        if j < (CHUNK // 32):
            pos0_v[pl.ds(j * 16, 16)] = posc
            tok0_v[pl.ds(j * 16, 16)] = tok
        else:
            jj = j - CHUNK // 32
            pos1_v[pl.ds(jj * 16, 16)] = posc
            tok1_v[pl.ds(jj * 16, 16)] = tok

    @pl.when(cid == 0)
    def _write_inv():
        pltpu.sync_copy(inv_v, inv_hbm.at[pl.ds(sid * CHUNK, CHUNK)])

    pltpu.async_copy(tok0_v, st_hbm.at[pos0_v], ssem).wait()
    pltpu.async_copy(tok1_v, st_hbm.at[pos1_v], ssem).wait()
    plsc.subcore_barrier()

    wid = sid * 2 + cid
    rows_per = NP // 32
    gbase = wid * rows_per
    pltpu.sync_copy(st_hbm.at[pl.ds(cid * NP + gbase, rows_per)], idx_v)
    for i in range(rows_per // 16):
        u = idx_v[pl.ds(i * 16, 16)]
        idx_v[pl.ds(i * 16, 16)] = jnp.clip(u, 0, T - 1)

    bufs = (a_v, b_v, c_v)
    gsems = (gsa, gsb, gsc)
    wsems = (wsa, wsb, wsc)

    def gact(c):
        return gbase + c * _GCH < total_padded

    def gfire(c):
        @pl.when(gact(c))
        def _():
            pltpu.async_copy(
                x_hbm.at[idx_v.at[pl.ds(c * _GCH, _GCH)]], bufs[c % 3],
                gsems[c % 3])

    gfire(0)
    gfire(1)
    gfire(2)
    for c in range(_GN):
        @pl.when(gact(c))
        def _step(c=c):
            pltpu.make_async_copy(
                x_hbm.at[idx_v.at[pl.ds(c * _GCH, _GCH)]], bufs[c % 3],
                gsems[c % 3]).wait()
            pltpu.async_copy(
                bufs[c % 3], xs_hbm.at[pl.ds(gbase + c * _GCH, _GCH)],
                wsems[c % 3]).wait()
        if c + 3 < _GN:
            gfire(c + 3)


def _dispatch_gather(eid_flat, x):
    mesh = plsc.VectorSubcoreMesh(core_axis_name="c", subcore_axis_name="s")
    f = pl.kernel(
        _dg_body,
        out_type=[
            jax.ShapeDtypeStruct((NP, D), jnp.float32),
            jax.ShapeDtypeStruct((T * K,), jnp.int32),
            jax.ShapeDtypeStruct((32,), jnp.int32),
        ],
        mesh=mesh,
        compiler_params=pltpu.CompilerParams(needs_layout_passes=False),
        scratch_types=[
            pltpu.VMEM((CHUNK,), jnp.int32),
            pltpu.VMEM((16,), jnp.int32),
            pltpu.VMEM((NSUB, 16), jnp.int32),
            pltpu.VMEM((CHUNK,), jnp.int32),
            pltpu.VMEM((CHUNK // 2,), jnp.int32),
            pltpu.VMEM((CHUNK // 2,), jnp.int32),
            pltpu.VMEM((CHUNK // 2,), jnp.int32),
            pltpu.VMEM((CHUNK // 2,), jnp.int32),
            pltpu.VMEM((32,), jnp.int32),
            pltpu.VMEM((16,), jnp.int32),
            pltpu.VMEM((NP // 32,), jnp.int32),
            pltpu.VMEM((_GCH, D), jnp.float32),
            pltpu.VMEM((_GCH, D), jnp.float32),
            pltpu.VMEM((_GCH, D), jnp.float32),
            pltpu.HBM((NSUB, 16), jnp.int32),
            pltpu.HBM((2 * NP,), jnp.int32),
            pltpu.SemaphoreType.DMA,
            pltpu.SemaphoreType.DMA,
            pltpu.SemaphoreType.DMA,
            pltpu.SemaphoreType.DMA,
            pltpu.SemaphoreType.DMA,
            pltpu.SemaphoreType.DMA,
            pltpu.SemaphoreType.DMA,
        ],
    )
    return f(eid_flat, x)


def _gemm_body(be_ref, xs_ref, w13_ref, w2_ref, y_ref):
    b = pl.program_id(0)
    e = be_ref[b]

    @pl.when(e >= 0)
    def _compute():
        x = xs_ref[...].astype(jnp.bfloat16)
        h = lax.dot_general(x, w13_ref[0].astype(jnp.bfloat16),
                            (((1,), (1,)), ((), ())),
                            preferred_element_type=jnp.float32)
        gate = h[:, :D]
        up = h[:, D:]
        act = (gate * lax.logistic(gate) * up).astype(jnp.bfloat16)
        y_ref[...] = lax.dot_general(act, w2_ref[0].astype(jnp.bfloat16),
                                     (((1,), (1,)), ((), ())),
                                     preferred_element_type=jnp.float32)


def _gemm(block_expert, x_s, w13, w2):
    grid_spec = pltpu.PrefetchScalarGridSpec(
        num_scalar_prefetch=1,
        grid=(NBLK,),
        in_specs=[
            pl.BlockSpec((BLK, D), lambda b, be: (b, 0)),
            pl.BlockSpec((1, 2 * D, D),
                         lambda b, be: (jnp.maximum(be[b], 0), 0, 0)),
            pl.BlockSpec((1, D, D),
                         lambda b, be: (jnp.maximum(be[b], 0), 0, 0)),
        ],
        out_specs=pl.BlockSpec((BLK, D), lambda b, be: (b, 0)),
    )
    return pl.pallas_call(
        _gemm_body,
        grid_spec=grid_spec,
        out_shape=jax.ShapeDtypeStruct((NP, D), jnp.float32),
    )(block_expert, x_s, w13, w2)


def _combine_body(inv_hbm, w0_hbm, w1_hbm, y_hbm, out_hbm,
                  i00_v, i01_v, i10_v, i11_v, w0_v, w1_v, a_v, b_v, sem):
    wid = lax.axis_index("s") * 2 + lax.axis_index("c")
    t0 = wid * (T // 32)
    pltpu.sync_copy(inv_hbm.at[pl.ds(t0, 32)], i00_v)
    pltpu.sync_copy(inv_hbm.at[pl.ds(t0 + 32, 32)], i01_v)
    pltpu.sync_copy(inv_hbm.at[pl.ds(T + t0, 32)], i10_v)
    pltpu.sync_copy(inv_hbm.at[pl.ds(T + t0 + 32, 32)], i11_v)
    pltpu.sync_copy(w0_hbm.at[pl.ds(t0, 64)], w0_v)
    pltpu.sync_copy(w1_hbm.at[pl.ds(t0, 64)], w1_v)
    for c in range(2):
        ia = i00_v if c == 0 else i01_v
        ib = i10_v if c == 0 else i11_v
        pltpu.async_copy(y_hbm.at[ia], a_v, sem).wait()
        pltpu.async_copy(y_hbm.at[ib], b_v, sem).wait()

        def _token(i, _):
            ti = c * 32 + i
            li = jnp.full((16,), ti, jnp.int32)
            s0 = plsc.load_gather(w0_v, [li])
            s1 = plsc.load_gather(w1_v, [li])

            def _col(j, _):
                cj = pl.multiple_of(j * 16, 16)
                av = a_v[i, pl.ds(cj, 16)]
                bv = b_v[i, pl.ds(cj, 16)]
                a_v[i, pl.ds(cj, 16)] = s0 * av + s1 * bv
                return 0

            lax.fori_loop(0, D // 16, _col, 0)
            return 0

        lax.fori_loop(0, 32, _token, 0)
        pltpu.sync_copy(a_v, out_hbm.at[pl.ds(t0 + c * 32, 32)])


def _combine(inv, w0, w1, y):
    mesh = plsc.VectorSubcoreMesh(core_axis_name="c", subcore_axis_name="s")
    f = pl.kernel(
        _combine_body,
        out_type=jax.ShapeDtypeStruct((T, D), jnp.float32),
        mesh=mesh,
        compiler_params=pltpu.CompilerParams(needs_layout_passes=False),
        scratch_types=[
            pltpu.VMEM((32,), jnp.int32),
            pltpu.VMEM((32,), jnp.int32),
            pltpu.VMEM((32,), jnp.int32),
            pltpu.VMEM((32,), jnp.int32),
            pltpu.VMEM((64,), jnp.float32),
            pltpu.VMEM((64,), jnp.float32),
            pltpu.VMEM((32, D), jnp.float32),
            pltpu.VMEM((32, D), jnp.float32),
            pltpu.SemaphoreType.DMA,
        ],
    )
    return f(inv, w0, w1, y)


def kernel(hidden_states, w_router, w13, w2):
    logits, i1, i2, wt1, wt2 = _router(hidden_states, w_router)
    eid_flat = jnp.concatenate([i1, i2], axis=0).reshape(T * K)
    x_s, inv, block_expert = _dispatch_gather(eid_flat, hidden_states)
    y = _gemm(block_expert[:NBLK], x_s, w13, w2)
    out = _combine(inv, wt1.reshape(T), wt2.reshape(T), y)
    return out, logits
